# baseline (speedup 1.0000x reference)
import functools

import jax
import jax.numpy as jnp
from jax import lax
from jax.experimental import pallas as pl
from jax.experimental.pallas import tpu as pltpu
from jax.experimental.pallas import tpu_sc as plsc

_B = 16384
_D = 64
_NC, _NS, _L = 2, 16, 16
_NW = _NC * _NS
_V = 100000
_XCHUNK = 4096
_NXCHUNK = _B // _XCHUNK
_UNROLL = 8

_mesh = plsc.VectorSubcoreMesh(core_axis_name="c", subcore_axis_name="s")


def _make_half(f_base):
    @functools.partial(
        pl.kernel,
        out_type=jax.ShapeDtypeStruct((_NW, _L), jnp.float32),
        mesh=_mesh,
        scratch_types=[
            pltpu.VMEM((_B,), jnp.int32),
            pltpu.VMEM((1, _V), jnp.float32),
            pltpu.VMEM((2, _XCHUNK), jnp.float32),
            pltpu.VMEM((_L,), jnp.float32),
            pltpu.SemaphoreType.DMA,
            pltpu.SemaphoreType.DMA,
        ],
        compiler_params=pltpu.CompilerParams(skip_device_barrier=True,
                                             needs_layout_passes=False),
    )
    def _sc_partial(xt_hbm, idx_hbm, tabt_hbm, out_hbm,
                    idx_v, crow_v, x_v, acc_v, csem, xsem):
        wid = lax.axis_index("c") * _NS + lax.axis_index("s")
        f = f_base + wid

        row_copy = pltpu.async_copy(tabt_hbm.at[f], crow_v.at[0], csem)
        pltpu.sync_copy(idx_hbm, idx_v)
        crow_flat = crow_v.at[0]

        x_copies = [pltpu.async_copy(
            xt_hbm.at[f, pl.ds(0, _XCHUNK)], x_v.at[0], xsem)]
        row_copy.wait()
        acc = jnp.zeros((_L,), jnp.float32)
        for cx in range(_NXCHUNK):
            if cx + 1 < _NXCHUNK:
                x_copies.append(pltpu.async_copy(
                    xt_hbm.at[f, pl.ds((cx + 1) * _XCHUNK, _XCHUNK)],
                    x_v.at[(cx + 1) % 2], xsem))
            x_copies[cx].wait()
            xbuf = cx % 2

            def group(g, acc, cx=cx, xbuf=xbuf):
                o = g * _L
                idxv = idx_v[pl.ds(cx * _XCHUNK + o, _L)]
                cv = plsc.load_gather(crow_flat, [idxv])
                xv = x_v[xbuf, pl.ds(o, _L)]
                d = xv - cv
                return acc + d * d

            acc = plsc.parallel_loop(0, _XCHUNK // _L, step=1,
                                     unroll=_UNROLL, carry=acc)(group)

        acc_v[...] = acc
        pltpu.sync_copy(acc_v, out_hbm.at[wid])

    return _sc_partial


_HALF0 = _make_half(0)
_HALF1 = _make_half(_NW)


def kernel(latent_X, cluster_id, clusters):
    idx = cluster_id.astype(jnp.int32)
    xt, tabt = latent_X.T, clusters.T
    p0 = _HALF0(xt, idx, tabt)
    p1 = _HALF1(xt, idx, tabt)
    return 0.5 * (jnp.sum(p0) + jnp.sum(p1))


# speedup vs baseline: 1.2007x; 1.2007x over previous
import functools

import jax
import jax.numpy as jnp
from jax import lax
from jax.experimental import pallas as pl
from jax.experimental.pallas import tpu as pltpu
from jax.experimental.pallas import tpu_sc as plsc

_B = 16384
_D = 64
_NC, _NS, _L = 2, 16, 16
_NW = _NC * _NS
_FPW = _D // _NW
_V = 100000
_XCHUNK = 4096
_NXCHUNK = _B // _XCHUNK

_mesh = plsc.VectorSubcoreMesh(core_axis_name="c", subcore_axis_name="s")


@functools.partial(
    pl.kernel,
    out_type=jax.ShapeDtypeStruct((_NW, _L), jnp.float32),
    mesh=_mesh,
    scratch_types=[
        pltpu.VMEM((_B,), jnp.int32),
        pltpu.VMEM((1, _V), jnp.float32),
        pltpu.VMEM((2, _XCHUNK), jnp.float32),
        pltpu.VMEM((_L,), jnp.float32),
        pltpu.SemaphoreType.DMA,
        pltpu.SemaphoreType.DMA,
    ],
    compiler_params=pltpu.CompilerParams(skip_device_barrier=True,
                                         needs_layout_passes=False),
)
def _sc_partial(xt_hbm, idx_hbm, tabt_hbm, out_hbm,
                idx_v, crow_v, x_v, acc_v, csem, xsem):
    wid = lax.axis_index("c") * _NS + lax.axis_index("s")
    f0 = wid * _FPW

    row_copy = pltpu.async_copy(tabt_hbm.at[f0], crow_v.at[0], csem)
    pltpu.sync_copy(idx_hbm, idx_v)
    crow_flat = crow_v.at[0]
    _UNROLL = 8

    acc = jnp.zeros((_L,), jnp.float32)
    for fi in range(_FPW):
        x_copies = [pltpu.async_copy(
            xt_hbm.at[f0 + fi, pl.ds(0, _XCHUNK)], x_v.at[0], xsem)]
        row_copy.wait()
        for cx in range(_NXCHUNK):
            if cx + 1 < _NXCHUNK:
                x_copies.append(pltpu.async_copy(
                    xt_hbm.at[f0 + fi, pl.ds((cx + 1) * _XCHUNK, _XCHUNK)],
                    x_v.at[(cx + 1) % 2], xsem))
            x_copies[cx].wait()
            xbuf = cx % 2

            def group(g, acc, cx=cx, xbuf=xbuf):
                o = g * _L
                idxv = idx_v[pl.ds(cx * _XCHUNK + o, _L)]
                cv = plsc.load_gather(crow_flat, [idxv])
                xv = x_v[xbuf, pl.ds(o, _L)]
                d = xv - cv
                return acc + d * d

            acc = plsc.parallel_loop(0, _XCHUNK // _L, step=1,
                                     unroll=_UNROLL, carry=acc)(group)
        if fi + 1 < _FPW:
            row_copy = pltpu.async_copy(tabt_hbm.at[f0 + fi + 1],
                                        crow_v.at[0], csem)

    acc_v[...] = acc
    pltpu.sync_copy(acc_v, out_hbm.at[wid])


def kernel(latent_X, cluster_id, clusters):
    idx = cluster_id.astype(jnp.int32)
    partials = _sc_partial(latent_X.T, idx, clusters.T)
    return 0.5 * jnp.sum(partials)
